# baseline (speedup 1.0000x reference)
import jax
import jax.numpy as jnp
from jax.experimental import pallas as pl
from jax.experimental.pallas import tpu as pltpu

_EPS = 1e-3
_BLOCK_N = 512


def _bn_kernel(nvv_ref, x_ref, gamma_ref, beta_ref, mean_ref, var_ref, out_ref):
    b = pl.program_id(0)
    i = pl.program_id(1)
    nv = nvv_ref[b]
    base = i * _BLOCK_N

    block_n = x_ref.shape[1]
    scale = gamma_ref[0] * jax.lax.rsqrt(var_ref[0] + _EPS)
    bias = beta_ref[0] - mean_ref[0] * scale
    x = x_ref[0]
    y = x * scale[None, :] + bias[None, :]
    row = jax.lax.broadcasted_iota(jnp.int32, (block_n, 1), 0)
    mask = row < (nv - base)
    out_ref[0] = jnp.where(mask, y, jnp.zeros_like(y))


def kernel(voxel_features, num_valid_voxels, gamma, beta, moving_mean, moving_var):
    B, N, FD = voxel_features.shape
    nb = N // _BLOCK_N

    def x_map(b, i, nvv):
        last = jnp.maximum(pl.cdiv(nvv[b], _BLOCK_N) - 1, 0)
        return (b, jnp.minimum(i, last), 0)

    def param_map(b, i, nvv):
        return (0, 0)

    grid_spec = pltpu.PrefetchScalarGridSpec(
        num_scalar_prefetch=1,
        grid=(B, nb),
        in_specs=[
            pl.BlockSpec((1, _BLOCK_N, FD), x_map),
            pl.BlockSpec((1, FD), param_map),
            pl.BlockSpec((1, FD), param_map),
            pl.BlockSpec((1, FD), param_map),
            pl.BlockSpec((1, FD), param_map),
        ],
        out_specs=pl.BlockSpec((1, _BLOCK_N, FD), lambda b, i, nvv: (b, i, 0)),
    )

    return pl.pallas_call(
        _bn_kernel,
        grid_spec=grid_spec,
        out_shape=jax.ShapeDtypeStruct((B, N, FD), voxel_features.dtype),
    )(
        num_valid_voxels,
        voxel_features,
        gamma.reshape(1, FD),
        beta.reshape(1, FD),
        moving_mean.reshape(1, FD),
        moving_var.reshape(1, FD),
    )


# speedup vs baseline: 1.0010x; 1.0010x over previous
import jax
import jax.numpy as jnp
from jax.experimental import pallas as pl
from jax.experimental.pallas import tpu as pltpu

_EPS = 1e-3
_BLOCK_N = 512


def _bn_kernel(nvv_ref, x_ref, gamma_ref, beta_ref, mean_ref, var_ref, out_ref):
    b = pl.program_id(0)
    i = pl.program_id(1)
    nv = nvv_ref[b]
    base = i * _BLOCK_N

    block_n = x_ref.shape[1]
    scale = gamma_ref[0] * jax.lax.rsqrt(var_ref[0] + _EPS)
    bias = beta_ref[0] - mean_ref[0] * scale
    x = x_ref[0]
    y = x * scale[None, :] + bias[None, :]
    row = jax.lax.broadcasted_iota(jnp.int32, (block_n, 1), 0)
    mask = row < (nv - base)
    out_ref[0] = jnp.where(mask, y, jnp.zeros_like(y))


def kernel(voxel_features, num_valid_voxels, gamma, beta, moving_mean, moving_var):
    B, N, FD = voxel_features.shape
    nb = N // _BLOCK_N

    def x_map(b, i, nvv):
        last = jnp.maximum(pl.cdiv(nvv[b], _BLOCK_N) - 1, 0)
        return (b, jnp.minimum(i, last), 0)

    def param_map(b, i, nvv):
        return (0, 0)

    grid_spec = pltpu.PrefetchScalarGridSpec(
        num_scalar_prefetch=1,
        grid=(B, nb),
        in_specs=[
            pl.BlockSpec((1, _BLOCK_N, FD), x_map),
            pl.BlockSpec((1, FD), param_map),
            pl.BlockSpec((1, FD), param_map),
            pl.BlockSpec((1, FD), param_map),
            pl.BlockSpec((1, FD), param_map),
        ],
        out_specs=pl.BlockSpec((1, _BLOCK_N, FD), lambda b, i, nvv: (b, i, 0)),
    )

    return pl.pallas_call(
        _bn_kernel,
        grid_spec=grid_spec,
        out_shape=jax.ShapeDtypeStruct((B, N, FD), voxel_features.dtype),
        compiler_params=pltpu.CompilerParams(
            dimension_semantics=("parallel", "arbitrary"),
        ),
    )(
        num_valid_voxels,
        voxel_features,
        gamma.reshape(1, FD),
        beta.reshape(1, FD),
        moving_mean.reshape(1, FD),
        moving_var.reshape(1, FD),
    )


# speedup vs baseline: 1.8462x; 1.8444x over previous
import jax
import jax.numpy as jnp
from jax.experimental import pallas as pl
from jax.experimental.pallas import tpu as pltpu

_EPS = 1e-3
_BLOCK_N = 2048


def _bn_kernel(nvv_ref, x_ref, gamma_ref, beta_ref, mean_ref, var_ref, out_ref):
    b = pl.program_id(0)
    i = pl.program_id(1)
    nv = nvv_ref[b]
    base = i * _BLOCK_N

    block_n = x_ref.shape[1]
    scale = gamma_ref[0] * jax.lax.rsqrt(var_ref[0] + _EPS)
    bias = beta_ref[0] - mean_ref[0] * scale
    x = x_ref[0]
    y = x * scale[None, :] + bias[None, :]
    row = jax.lax.broadcasted_iota(jnp.int32, (block_n, 1), 0)
    mask = row < (nv - base)
    out_ref[0] = jnp.where(mask, y, jnp.zeros_like(y))


def kernel(voxel_features, num_valid_voxels, gamma, beta, moving_mean, moving_var):
    B, N, FD = voxel_features.shape
    nb = N // _BLOCK_N

    def x_map(b, i, nvv):
        last = jnp.maximum(pl.cdiv(nvv[b], _BLOCK_N) - 1, 0)
        return (b, jnp.minimum(i, last), 0)

    def param_map(b, i, nvv):
        return (0, 0)

    grid_spec = pltpu.PrefetchScalarGridSpec(
        num_scalar_prefetch=1,
        grid=(B, nb),
        in_specs=[
            pl.BlockSpec((1, _BLOCK_N, FD), x_map),
            pl.BlockSpec((1, FD), param_map),
            pl.BlockSpec((1, FD), param_map),
            pl.BlockSpec((1, FD), param_map),
            pl.BlockSpec((1, FD), param_map),
        ],
        out_specs=pl.BlockSpec((1, _BLOCK_N, FD), lambda b, i, nvv: (b, i, 0)),
    )

    return pl.pallas_call(
        _bn_kernel,
        grid_spec=grid_spec,
        out_shape=jax.ShapeDtypeStruct((B, N, FD), voxel_features.dtype),
        compiler_params=pltpu.CompilerParams(
            dimension_semantics=("parallel", "arbitrary"),
        ),
    )(
        num_valid_voxels,
        voxel_features,
        gamma.reshape(1, FD),
        beta.reshape(1, FD),
        moving_mean.reshape(1, FD),
        moving_var.reshape(1, FD),
    )
